# baseline (speedup 1.0000x reference)
import jax
import jax.numpy as jnp
from jax import lax
from jax.experimental import pallas as pl
from jax.experimental.pallas import tpu as pltpu, tpu_sc as plsc

_EMBED_DIM = 64


def _row_gather_body(table_hbm, idx_hbm, out_hbm, idx_v, row_v, sem):
    cid = lax.axis_index("c")
    sid = lax.axis_index("s")

    @pl.when(jnp.logical_and(cid == 0, sid == 0))
    def _():
        pltpu.sync_copy(idx_hbm, idx_v)
        pltpu.async_copy(table_hbm.at[idx_v], row_v, sem).wait()
        pltpu.sync_copy(row_v, out_hbm)


def kernel(table, domain_id):
    idx = jnp.asarray(domain_id, jnp.int32).reshape((1,))
    mesh = plsc.VectorSubcoreMesh(core_axis_name="c", subcore_axis_name="s")
    gather = pl.kernel(
        _row_gather_body,
        mesh=mesh,
        out_type=jax.ShapeDtypeStruct((1, _EMBED_DIM), jnp.float32),
        scratch_types=[
            pltpu.VMEM((1,), jnp.int32),
            pltpu.VMEM((1, _EMBED_DIM), jnp.float32),
            pltpu.SemaphoreType.DMA,
        ],
        compiler_params=pltpu.CompilerParams(use_tc_tiling_on_sc=False),
    )
    out = gather(table, idx)
    return out.reshape((_EMBED_DIM,))


# speedup vs baseline: 1.7327x; 1.7327x over previous
import jax
import jax.numpy as jnp
from jax import lax
from jax.experimental import pallas as pl
from jax.experimental.pallas import tpu as pltpu, tpu_sc as plsc

_EMBED_DIM = 64


def _row_gather_body(table_hbm, idx_hbm, out_hbm, idx_s):
    @pl.when(lax.axis_index("c") == 0)
    def _():
        pltpu.sync_copy(idx_hbm, idx_s)
        i = idx_s[0]
        pltpu.sync_copy(table_hbm.at[pl.ds(i, 1), :], out_hbm)


def kernel(table, domain_id):
    idx = jnp.asarray(domain_id, jnp.int32).reshape((1,))
    mesh = plsc.ScalarSubcoreMesh(axis_name="c", num_cores=2)
    gather = pl.kernel(
        _row_gather_body,
        mesh=mesh,
        out_type=jax.ShapeDtypeStruct((1, _EMBED_DIM), jnp.float32),
        scratch_types=[
            pltpu.SMEM((1,), jnp.int32),
        ],
    )
    out = gather(table, idx)
    return out.reshape((_EMBED_DIM,))
